# baseline (speedup 1.0000x reference)
import jax
import jax.numpy as jnp
from jax import lax
from jax.experimental import pallas as pl
from jax.experimental.pallas import tpu as pltpu

_VMEM_LIMIT = 64 * 1024 * 1024


def _pick_tile(n, target):
    t = min(target, n)
    while n % t:
        t -= 1
    return t


def _fused_kernel(emb_ref, mask_ref, wp_ref, bp_ref, wn_ref, wv_ref,
                  aux_ref, bn_ref, bv_ref,
                  loss_ref, nextp_ref, prevp_ref,
                  enc_all, v_all, num_acc, den_acc):
    step = pl.program_id(0)
    tE_ = emb_ref.shape[0]
    tN = nextp_ref.shape[0]
    N, H = enc_all.shape
    GE = N // tE_

    @pl.when(step == 0)
    def _():
        num_acc[...] = jnp.zeros_like(num_acc)
        den_acc[...] = jnp.zeros_like(den_acc)

    @pl.when(step < GE)
    def _encode():
        mask = mask_ref[...]
        denom = jnp.maximum(jnp.sum(mask, axis=1, keepdims=True), 1.0)
        pooled = jnp.sum(emb_ref[...] * mask[:, :, None], axis=1) / denom
        enc = (jnp.dot(pooled, wp_ref[...], preferred_element_type=jnp.float32)
               + bp_ref[...])
        row = pl.multiple_of(step * tE_, 8)
        enc_all[pl.ds(row, tE_), :] = enc
        v_all[pl.ds(row, tE_), 0:H] = jnp.dot(
            enc, wn_ref[...], preferred_element_type=jnp.float32)
        v_all[pl.ds(row, tE_), H:2 * H] = jnp.dot(
            enc, wv_ref[...], preferred_element_type=jnp.float32)

    @pl.when(step >= GE)
    def _score():
        row = pl.multiple_of((step - GE) * tN, 8)
        v = v_all[pl.ds(row, tN), :]
        O = aux_ref.shape[1] - 3
        opts = aux_ref[:, 0:O]

        dims = (((1,), (1,)), ((), ()))
        v_next = v[:, :H]
        v_prev = v[:, H:]
        s_next = lax.dot_general(v_next, enc_all[...], dims,
                                 preferred_element_type=jnp.float32)
        s_prev = lax.dot_general(v_prev, enc_all[...], dims,
                                 preferred_element_type=jnp.float32)

        CW = min(128, N)
        q_all = opts // CW
        r_all = opts % CW
        vals_n, vals_p = [], []
        for q in range(N // CW):
            g_n = jnp.take_along_axis(s_next[:, q * CW:(q + 1) * CW],
                                      r_all, axis=1)
            g_p = jnp.take_along_axis(s_prev[:, q * CW:(q + 1) * CW],
                                      r_all, axis=1)
            hit = q_all == q
            vals_n.append(jnp.where(hit, g_n, 0.0))
            vals_p.append(jnp.where(hit, g_p, 0.0))
        def _tree(vals):
            while len(vals) > 1:
                nxt = [vals[i] + vals[i + 1]
                       for i in range(0, len(vals) - 1, 2)]
                if len(vals) % 2:
                    nxt.append(vals[-1])
                vals = nxt
            return vals
        vals_n = _tree(vals_n)
        vals_p = _tree(vals_p)
        next_logits = vals_n[0] + bn_ref[0]
        prev_logits = vals_p[0] + bv_ref[0]

        def gold_nll(x, gold):
            m = jnp.max(x, axis=-1, keepdims=True)
            s = x - m
            lse = jnp.log(jnp.sum(jnp.exp(s), axis=-1, keepdims=True))
            return lse - jnp.take_along_axis(s, gold, axis=1)

        losses = (gold_nll(next_logits, aux_ref[:, O:O + 1])
                  + gold_nll(prev_logits, aux_ref[:, O + 1:O + 2]))

        cm = aux_ref[:, O + 2:O + 3].astype(jnp.float32)
        num_acc[...] += jnp.sum(losses * cm).reshape(1, 1)
        den_acc[...] += jnp.sum(cm).reshape(1, 1)

        col_o = lax.broadcasted_iota(jnp.int32, (tN, O), 1)

        def argmax_first(x):
            m = jnp.max(x, axis=-1, keepdims=True)
            idx = jnp.where(x == m, col_o, jnp.int32(O))
            return jnp.min(idx, axis=-1, keepdims=True)

        nextp_ref[...] = argmax_first(next_logits)
        prevp_ref[...] = argmax_first(prev_logits)

    @pl.when(step == pl.num_programs(0) - 1)
    def _():
        loss_ref[...] = num_acc[...] / (2.0 * den_acc[...])


def kernel(embeddings, input_mask, conversation_mask, options_tensor,
           gold_next, gold_prev, w_proj, b_proj, w_next, b_next,
           w_prev, b_prev):
    N, T, E = embeddings.shape
    H = w_proj.shape[1]
    O = options_tensor.shape[1]

    aux = jnp.concatenate(
        [options_tensor.astype(jnp.int32),
         gold_next.reshape(N, 1).astype(jnp.int32),
         gold_prev.reshape(N, 1).astype(jnp.int32),
         conversation_mask.reshape(N, 1).astype(jnp.int32)], axis=1)

    tE_ = _pick_tile(N, 128)
    tN = _pick_tile(N, 256)
    GE = N // tE_
    GS = N // tN

    def enc_map(i):
        return (jnp.minimum(i, GE - 1), 0)

    def score_map(i):
        return (jnp.maximum(i - GE, 0), 0)

    loss, nextp, prevp = pl.pallas_call(
        _fused_kernel,
        out_shape=(jax.ShapeDtypeStruct((1, 1), jnp.float32),
                   jax.ShapeDtypeStruct((N, 1), jnp.int32),
                   jax.ShapeDtypeStruct((N, 1), jnp.int32)),
        grid=(GE + GS,),
        in_specs=[
            pl.BlockSpec((tE_, T, E),
                         lambda i: (jnp.minimum(i, GE - 1), 0, 0)),
            pl.BlockSpec((tE_, T), enc_map),
            pl.BlockSpec((E, H), lambda i: (0, 0)),
            pl.BlockSpec((1, H), lambda i: (0, 0)),
            pl.BlockSpec((H, H), lambda i: (0, 0)),
            pl.BlockSpec((H, H), lambda i: (0, 0)),
            pl.BlockSpec((tN, O + 3), score_map),
            pl.BlockSpec(memory_space=pltpu.MemorySpace.SMEM),
            pl.BlockSpec(memory_space=pltpu.MemorySpace.SMEM),
        ],
        out_specs=(pl.BlockSpec((1, 1), lambda i: (0, 0)),
                   pl.BlockSpec((tN, 1), score_map),
                   pl.BlockSpec((tN, 1), score_map)),
        scratch_shapes=[pltpu.VMEM((N, H), jnp.float32),
                        pltpu.VMEM((N, 2 * H), jnp.float32),
                        pltpu.VMEM((1, 1), jnp.float32),
                        pltpu.VMEM((1, 1), jnp.float32)],
        compiler_params=pltpu.CompilerParams(
            dimension_semantics=("arbitrary",),
            vmem_limit_bytes=_VMEM_LIMIT),
    )(embeddings, input_mask, w_proj, b_proj.reshape(1, H), w_next, w_prev,
      aux, b_next.astype(jnp.float32), b_prev.astype(jnp.float32))

    return loss[0, 0], (nextp[:, 0], prevp[:, 0])


# speedup vs baseline: 1.0670x; 1.0670x over previous
import jax
import jax.numpy as jnp
from jax import lax
from jax.experimental import pallas as pl
from jax.experimental.pallas import tpu as pltpu

_VMEM_LIMIT = 64 * 1024 * 1024


def _pick_tile(n, target):
    t = min(target, n)
    while n % t:
        t -= 1
    return t


def _fused_kernel(emb_ref, mask_ref, wp_ref, bp_ref, wn_ref, wv_ref,
                  aux_ref, bn_ref, bv_ref,
                  loss_ref, nextp_ref, prevp_ref,
                  enc_all, v_all, num_acc, den_acc):
    step = pl.program_id(0)
    tE_ = emb_ref.shape[0]
    tN = nextp_ref.shape[0]
    N, H = enc_all.shape
    GE = N // tE_

    @pl.when(step == 0)
    def _():
        num_acc[...] = jnp.zeros_like(num_acc)
        den_acc[...] = jnp.zeros_like(den_acc)

    @pl.when(step < GE)
    def _encode():
        mask = mask_ref[...]
        denom = jnp.maximum(jnp.sum(mask, axis=1, keepdims=True), 1.0)
        pooled = jnp.sum(emb_ref[...] * mask[:, :, None], axis=1) / denom
        enc = (jnp.dot(pooled, wp_ref[...], preferred_element_type=jnp.float32)
               + bp_ref[...])
        row = pl.multiple_of(step * tE_, 8)
        enc_all[pl.ds(row, tE_), :] = enc
        v_all[pl.ds(row, tE_), 0:H] = jnp.dot(
            enc, wn_ref[...], preferred_element_type=jnp.float32)
        v_all[pl.ds(row, tE_), H:2 * H] = jnp.dot(
            enc, wv_ref[...], preferred_element_type=jnp.float32)

    @pl.when(step >= GE)
    def _score():
        row = pl.multiple_of((step - GE) * tN, 8)
        v = v_all[pl.ds(row, tN), :]
        O = aux_ref.shape[1] - 3
        opts = aux_ref[:, 0:O]

        dims = (((1,), (1,)), ((), ()))
        v_next = v[:, :H]
        v_prev = v[:, H:]
        s_next = lax.dot_general(v_next, enc_all[...], dims,
                                 preferred_element_type=jnp.float32)
        s_prev = lax.dot_general(v_prev, enc_all[...], dims,
                                 preferred_element_type=jnp.float32)

        CW = min(128, N)
        q_all = opts // CW
        r_all = opts % CW
        vals_n, vals_p = [], []
        for q in range(N // CW):
            g_n = jnp.take_along_axis(s_next[:, q * CW:(q + 1) * CW],
                                      r_all, axis=1)
            g_p = jnp.take_along_axis(s_prev[:, q * CW:(q + 1) * CW],
                                      r_all, axis=1)
            hit = q_all == q
            vals_n.append(jnp.where(hit, g_n, 0.0))
            vals_p.append(jnp.where(hit, g_p, 0.0))
        def _tree(vals):
            while len(vals) > 1:
                nxt = [vals[i] + vals[i + 1]
                       for i in range(0, len(vals) - 1, 2)]
                if len(vals) % 2:
                    nxt.append(vals[-1])
                vals = nxt
            return vals
        vals_n = _tree(vals_n)
        vals_p = _tree(vals_p)
        next_logits = vals_n[0] + bn_ref[0]
        prev_logits = vals_p[0] + bv_ref[0]

        def gold_nll(x, gold):
            m = jnp.max(x, axis=-1, keepdims=True)
            s = x - m
            lse = jnp.log(jnp.sum(jnp.exp(s), axis=-1, keepdims=True))
            return lse - jnp.take_along_axis(s, gold, axis=1)

        losses = (gold_nll(next_logits, aux_ref[:, O:O + 1])
                  + gold_nll(prev_logits, aux_ref[:, O + 1:O + 2]))

        cm = aux_ref[:, O + 2:O + 3].astype(jnp.float32)
        num_acc[...] += jnp.sum(losses * cm).reshape(1, 1)
        den_acc[...] += jnp.sum(cm).reshape(1, 1)

        col_o = lax.broadcasted_iota(jnp.int32, (tN, O), 1)

        def argmax_first(x):
            m = jnp.max(x, axis=-1, keepdims=True)
            idx = jnp.where(x == m, col_o, jnp.int32(O))
            return jnp.min(idx, axis=-1, keepdims=True)

        nextp_ref[...] = argmax_first(next_logits)
        prevp_ref[...] = argmax_first(prev_logits)

    @pl.when(step == pl.num_programs(0) - 1)
    def _():
        loss_ref[...] = num_acc[...] / (2.0 * den_acc[...])


def kernel(embeddings, input_mask, conversation_mask, options_tensor,
           gold_next, gold_prev, w_proj, b_proj, w_next, b_next,
           w_prev, b_prev):
    N, T, E = embeddings.shape
    H = w_proj.shape[1]
    O = options_tensor.shape[1]

    aux = jnp.concatenate(
        [options_tensor.astype(jnp.int32),
         gold_next.reshape(N, 1).astype(jnp.int32),
         gold_prev.reshape(N, 1).astype(jnp.int32),
         conversation_mask.reshape(N, 1).astype(jnp.int32)], axis=1)

    tE_ = _pick_tile(N, 512)
    tN = _pick_tile(N, 256)
    GE = N // tE_
    GS = N // tN

    def enc_map(i):
        return (jnp.minimum(i, GE - 1), 0)

    def score_map(i):
        return (jnp.maximum(i - GE, 0), 0)

    loss, nextp, prevp = pl.pallas_call(
        _fused_kernel,
        out_shape=(jax.ShapeDtypeStruct((1, 1), jnp.float32),
                   jax.ShapeDtypeStruct((N, 1), jnp.int32),
                   jax.ShapeDtypeStruct((N, 1), jnp.int32)),
        grid=(GE + GS,),
        in_specs=[
            pl.BlockSpec((tE_, T, E),
                         lambda i: (jnp.minimum(i, GE - 1), 0, 0)),
            pl.BlockSpec((tE_, T), enc_map),
            pl.BlockSpec((E, H), lambda i: (0, 0)),
            pl.BlockSpec((1, H), lambda i: (0, 0)),
            pl.BlockSpec((H, H), lambda i: (0, 0)),
            pl.BlockSpec((H, H), lambda i: (0, 0)),
            pl.BlockSpec((tN, O + 3), score_map),
            pl.BlockSpec(memory_space=pltpu.MemorySpace.SMEM),
            pl.BlockSpec(memory_space=pltpu.MemorySpace.SMEM),
        ],
        out_specs=(pl.BlockSpec((1, 1), lambda i: (0, 0)),
                   pl.BlockSpec((tN, 1), score_map),
                   pl.BlockSpec((tN, 1), score_map)),
        scratch_shapes=[pltpu.VMEM((N, H), jnp.float32),
                        pltpu.VMEM((N, 2 * H), jnp.float32),
                        pltpu.VMEM((1, 1), jnp.float32),
                        pltpu.VMEM((1, 1), jnp.float32)],
        compiler_params=pltpu.CompilerParams(
            dimension_semantics=("arbitrary",),
            vmem_limit_bytes=_VMEM_LIMIT),
    )(embeddings, input_mask, w_proj, b_proj.reshape(1, H), w_next, w_prev,
      aux, b_next.astype(jnp.float32), b_prev.astype(jnp.float32))

    return loss[0, 0], (nextp[:, 0], prevp[:, 0])


# speedup vs baseline: 1.1098x; 1.0401x over previous
import jax
import jax.numpy as jnp
from jax import lax
from jax.experimental import pallas as pl
from jax.experimental.pallas import tpu as pltpu

_VMEM_LIMIT = 64 * 1024 * 1024


def _pick_tile(n, target):
    t = min(target, n)
    while n % t:
        t -= 1
    return t


def _fused_kernel(emb_ref, mask_ref, wp_ref, bp_ref, wn_ref, wv_ref,
                  aux_ref, bn_ref, bv_ref,
                  loss_ref, nextp_ref, prevp_ref,
                  enc_all, v_all, num_acc, den_acc):
    step = pl.program_id(0)
    tE_ = emb_ref.shape[0]
    tN = nextp_ref.shape[0]
    N, H = enc_all.shape
    GE = N // tE_

    @pl.when(step == 0)
    def _():
        num_acc[...] = jnp.zeros_like(num_acc)
        den_acc[...] = jnp.zeros_like(den_acc)

    @pl.when(step < GE)
    def _encode():
        mask = mask_ref[...]
        denom = jnp.maximum(jnp.sum(mask, axis=1, keepdims=True), 1.0)
        pooled = jnp.sum(emb_ref[...] * mask[:, :, None], axis=1) / denom
        enc = (jnp.dot(pooled, wp_ref[...], preferred_element_type=jnp.float32)
               + bp_ref[...])
        row = pl.multiple_of(step * tE_, 8)
        enc_all[pl.ds(row, tE_), :] = enc
        v_all[pl.ds(row, tE_), 0:H] = jnp.dot(
            enc, wn_ref[...], preferred_element_type=jnp.float32)
        v_all[pl.ds(row, tE_), H:2 * H] = jnp.dot(
            enc, wv_ref[...], preferred_element_type=jnp.float32)

    @pl.when(step >= GE)
    def _score():
        row = pl.multiple_of((step - GE) * tN, 8)
        v = v_all[pl.ds(row, tN), :]
        O = aux_ref.shape[1] - 3
        opts = aux_ref[:, 0:O]

        dims = (((1,), (1,)), ((), ()))
        v_next = v[:, :H]
        v_prev = v[:, H:]
        s_next = lax.dot_general(v_next, enc_all[...], dims,
                                 preferred_element_type=jnp.float32)
        s_prev = lax.dot_general(v_prev, enc_all[...], dims,
                                 preferred_element_type=jnp.float32)

        CW = min(128, N)
        q_all = opts // CW
        r_all = opts % CW
        vals_n, vals_p = [], []
        for q in range(N // CW):
            g_n = jnp.take_along_axis(s_next[:, q * CW:(q + 1) * CW],
                                      r_all, axis=1)
            g_p = jnp.take_along_axis(s_prev[:, q * CW:(q + 1) * CW],
                                      r_all, axis=1)
            hit = q_all == q
            vals_n.append(jnp.where(hit, g_n, 0.0))
            vals_p.append(jnp.where(hit, g_p, 0.0))
        def _tree(vals):
            while len(vals) > 1:
                nxt = [vals[i] + vals[i + 1]
                       for i in range(0, len(vals) - 1, 2)]
                if len(vals) % 2:
                    nxt.append(vals[-1])
                vals = nxt
            return vals
        vals_n = _tree(vals_n)
        vals_p = _tree(vals_p)
        next_logits = vals_n[0] + bn_ref[0]
        prev_logits = vals_p[0] + bv_ref[0]

        def gold_nll(x, gold):
            m = jnp.max(x, axis=-1, keepdims=True)
            s = x - m
            lse = jnp.log(jnp.sum(jnp.exp(s), axis=-1, keepdims=True))
            return lse - jnp.take_along_axis(s, gold, axis=1)

        losses = (gold_nll(next_logits, aux_ref[:, O:O + 1])
                  + gold_nll(prev_logits, aux_ref[:, O + 1:O + 2]))

        cm = aux_ref[:, O + 2:O + 3].astype(jnp.float32)
        num_acc[...] += jnp.sum(losses * cm).reshape(1, 1)
        den_acc[...] += jnp.sum(cm).reshape(1, 1)

        col_o = lax.broadcasted_iota(jnp.int32, (tN, O), 1)

        def argmax_first(x):
            m = jnp.max(x, axis=-1, keepdims=True)
            idx = jnp.where(x == m, col_o, jnp.int32(O))
            return jnp.min(idx, axis=-1, keepdims=True)

        nextp_ref[...] = argmax_first(next_logits)
        prevp_ref[...] = argmax_first(prev_logits)

    @pl.when(step == pl.num_programs(0) - 1)
    def _():
        loss_ref[...] = num_acc[...] / (2.0 * den_acc[...])


def kernel(embeddings, input_mask, conversation_mask, options_tensor,
           gold_next, gold_prev, w_proj, b_proj, w_next, b_next,
           w_prev, b_prev):
    N, T, E = embeddings.shape
    H = w_proj.shape[1]
    O = options_tensor.shape[1]

    aux = jnp.concatenate(
        [options_tensor.astype(jnp.int32),
         gold_next.reshape(N, 1).astype(jnp.int32),
         gold_prev.reshape(N, 1).astype(jnp.int32),
         conversation_mask.reshape(N, 1).astype(jnp.int32)], axis=1)

    tE_ = _pick_tile(N, 512)
    tN = _pick_tile(N, 512)
    GE = N // tE_
    GS = N // tN

    def enc_map(i):
        return (jnp.minimum(i, GE - 1), 0)

    def score_map(i):
        return (jnp.maximum(i - GE, 0), 0)

    loss, nextp, prevp = pl.pallas_call(
        _fused_kernel,
        out_shape=(jax.ShapeDtypeStruct((1, 1), jnp.float32),
                   jax.ShapeDtypeStruct((N, 1), jnp.int32),
                   jax.ShapeDtypeStruct((N, 1), jnp.int32)),
        grid=(GE + GS,),
        in_specs=[
            pl.BlockSpec((tE_, T, E),
                         lambda i: (jnp.minimum(i, GE - 1), 0, 0)),
            pl.BlockSpec((tE_, T), enc_map),
            pl.BlockSpec((E, H), lambda i: (0, 0)),
            pl.BlockSpec((1, H), lambda i: (0, 0)),
            pl.BlockSpec((H, H), lambda i: (0, 0)),
            pl.BlockSpec((H, H), lambda i: (0, 0)),
            pl.BlockSpec((tN, O + 3), score_map),
            pl.BlockSpec(memory_space=pltpu.MemorySpace.SMEM),
            pl.BlockSpec(memory_space=pltpu.MemorySpace.SMEM),
        ],
        out_specs=(pl.BlockSpec((1, 1), lambda i: (0, 0)),
                   pl.BlockSpec((tN, 1), score_map),
                   pl.BlockSpec((tN, 1), score_map)),
        scratch_shapes=[pltpu.VMEM((N, H), jnp.float32),
                        pltpu.VMEM((N, 2 * H), jnp.float32),
                        pltpu.VMEM((1, 1), jnp.float32),
                        pltpu.VMEM((1, 1), jnp.float32)],
        compiler_params=pltpu.CompilerParams(
            dimension_semantics=("arbitrary",),
            vmem_limit_bytes=_VMEM_LIMIT),
    )(embeddings, input_mask, w_proj, b_proj.reshape(1, H), w_next, w_prev,
      aux, b_next.astype(jnp.float32), b_prev.astype(jnp.float32))

    return loss[0, 0], (nextp[:, 0], prevp[:, 0])


# speedup vs baseline: 1.1252x; 1.0139x over previous
import jax
import jax.numpy as jnp
from jax import lax
from jax.experimental import pallas as pl
from jax.experimental.pallas import tpu as pltpu

_VMEM_LIMIT = 64 * 1024 * 1024


def _pick_tile(n, target):
    t = min(target, n)
    while n % t:
        t -= 1
    return t


def _fused_kernel(emb_ref, mask_ref, wp_ref, bp_ref, wn_ref, wv_ref,
                  aux_ref, bn_ref, bv_ref,
                  loss_ref, nextp_ref, prevp_ref,
                  enc_all, v_all, num_acc, den_acc):
    step = pl.program_id(0)
    tE_ = emb_ref.shape[0]
    tN = nextp_ref.shape[0]
    N, H = enc_all.shape
    GE = N // tE_

    @pl.when(step == 0)
    def _():
        num_acc[...] = jnp.zeros_like(num_acc)
        den_acc[...] = jnp.zeros_like(den_acc)

    @pl.when(step < GE)
    def _encode():
        mask = mask_ref[...]
        denom = jnp.maximum(jnp.sum(mask, axis=1, keepdims=True), 1.0)
        pooled = jnp.sum(emb_ref[...] * mask[:, :, None], axis=1) / denom
        enc = (jnp.dot(pooled, wp_ref[...], preferred_element_type=jnp.float32)
               + bp_ref[...])
        row = pl.multiple_of(step * tE_, 8)
        enc_all[pl.ds(row, tE_), :] = enc
        v_all[pl.ds(row, tE_), 0:H] = jnp.dot(
            enc, wn_ref[...], preferred_element_type=jnp.float32)
        v_all[pl.ds(row, tE_), H:2 * H] = jnp.dot(
            enc, wv_ref[...], preferred_element_type=jnp.float32)

    @pl.when(step >= GE)
    def _score():
        row = pl.multiple_of((step - GE) * tN, 8)
        v = v_all[pl.ds(row, tN), :]
        O = aux_ref.shape[1] - 3
        opts = aux_ref[:, 0:O]

        dims = (((1,), (1,)), ((), ()))
        v_next = v[:, :H]
        v_prev = v[:, H:]
        s_next = lax.dot_general(v_next, enc_all[...], dims,
                                 preferred_element_type=jnp.float32)
        s_prev = lax.dot_general(v_prev, enc_all[...], dims,
                                 preferred_element_type=jnp.float32)

        CW = min(128, N)
        q_all = opts // CW
        r_all = opts % CW
        vals_n, vals_p = [], []
        for q in range(N // CW):
            g_n = jnp.take_along_axis(s_next[:, q * CW:(q + 1) * CW],
                                      r_all, axis=1)
            g_p = jnp.take_along_axis(s_prev[:, q * CW:(q + 1) * CW],
                                      r_all, axis=1)
            hit = q_all == q
            vals_n.append(jnp.where(hit, g_n, 0.0))
            vals_p.append(jnp.where(hit, g_p, 0.0))
        def _tree(vals):
            while len(vals) > 1:
                nxt = [vals[i] + vals[i + 1]
                       for i in range(0, len(vals) - 1, 2)]
                if len(vals) % 2:
                    nxt.append(vals[-1])
                vals = nxt
            return vals
        vals_n = _tree(vals_n)
        vals_p = _tree(vals_p)
        next_logits = vals_n[0] + bn_ref[0]
        prev_logits = vals_p[0] + bv_ref[0]

        def gold_nll(x, gold):
            m = jnp.max(x, axis=-1, keepdims=True)
            s = x - m
            lse = jnp.log(jnp.sum(jnp.exp(s), axis=-1, keepdims=True))
            return lse - jnp.take_along_axis(s, gold, axis=1)

        losses = (gold_nll(next_logits, aux_ref[:, O:O + 1])
                  + gold_nll(prev_logits, aux_ref[:, O + 1:O + 2]))

        cm = aux_ref[:, O + 2:O + 3].astype(jnp.float32)
        num_acc[...] += jnp.sum(losses * cm).reshape(1, 1)
        den_acc[...] += jnp.sum(cm).reshape(1, 1)

        col_o = lax.broadcasted_iota(jnp.int32, (tN, O), 1)

        def argmax_first(x):
            m = jnp.max(x, axis=-1, keepdims=True)
            idx = jnp.where(x == m, col_o, jnp.int32(O))
            return jnp.min(idx, axis=-1, keepdims=True)

        nextp_ref[...] = argmax_first(next_logits)
        prevp_ref[...] = argmax_first(prev_logits)

    @pl.when(step == pl.num_programs(0) - 1)
    def _():
        loss_ref[...] = num_acc[...] / (2.0 * den_acc[...])


def kernel(embeddings, input_mask, conversation_mask, options_tensor,
           gold_next, gold_prev, w_proj, b_proj, w_next, b_next,
           w_prev, b_prev):
    N, T, E = embeddings.shape
    H = w_proj.shape[1]
    O = options_tensor.shape[1]

    aux = jnp.concatenate(
        [options_tensor.astype(jnp.int32),
         gold_next.reshape(N, 1).astype(jnp.int32),
         gold_prev.reshape(N, 1).astype(jnp.int32),
         conversation_mask.reshape(N, 1).astype(jnp.int32)], axis=1)

    tE_ = _pick_tile(N, 512)
    tN = _pick_tile(N, 1024)
    GE = N // tE_
    GS = N // tN

    def enc_map(i):
        return (jnp.minimum(i, GE - 1), 0)

    def score_map(i):
        return (jnp.maximum(i - GE, 0), 0)

    loss, nextp, prevp = pl.pallas_call(
        _fused_kernel,
        out_shape=(jax.ShapeDtypeStruct((1, 1), jnp.float32),
                   jax.ShapeDtypeStruct((N, 1), jnp.int32),
                   jax.ShapeDtypeStruct((N, 1), jnp.int32)),
        grid=(GE + GS,),
        in_specs=[
            pl.BlockSpec((tE_, T, E),
                         lambda i: (jnp.minimum(i, GE - 1), 0, 0)),
            pl.BlockSpec((tE_, T), enc_map),
            pl.BlockSpec((E, H), lambda i: (0, 0)),
            pl.BlockSpec((1, H), lambda i: (0, 0)),
            pl.BlockSpec((H, H), lambda i: (0, 0)),
            pl.BlockSpec((H, H), lambda i: (0, 0)),
            pl.BlockSpec((tN, O + 3), score_map),
            pl.BlockSpec(memory_space=pltpu.MemorySpace.SMEM),
            pl.BlockSpec(memory_space=pltpu.MemorySpace.SMEM),
        ],
        out_specs=(pl.BlockSpec((1, 1), lambda i: (0, 0)),
                   pl.BlockSpec((tN, 1), score_map),
                   pl.BlockSpec((tN, 1), score_map)),
        scratch_shapes=[pltpu.VMEM((N, H), jnp.float32),
                        pltpu.VMEM((N, 2 * H), jnp.float32),
                        pltpu.VMEM((1, 1), jnp.float32),
                        pltpu.VMEM((1, 1), jnp.float32)],
        compiler_params=pltpu.CompilerParams(
            dimension_semantics=("arbitrary",),
            vmem_limit_bytes=_VMEM_LIMIT),
    )(embeddings, input_mask, w_proj, b_proj.reshape(1, H), w_next, w_prev,
      aux, b_next.astype(jnp.float32), b_prev.astype(jnp.float32))

    return loss[0, 0], (nextp[:, 0], prevp[:, 0])
